# baseline (device time: 62477 ns/iter reference)
import jax
import jax.numpy as jnp
from jax import lax
from jax.experimental import pallas as pl
from jax.experimental.pallas import tpu as pltpu

N_DEV = 4
N_EXPERTS = 16
N_LOCAL_E = 4


def kernel(x, router_W, route_idx, expert_W, shared_W):
    n, d = x.shape
    _, _, h = expert_W.shape
    chunk = n // N_DEV

    def body(x_ref, router_ref, idx_ref, expW_ref, sharedW_ref,
             out_ref, partial_ref, comm_ref, send_sems, recv_sems):
        my_pos = lax.axis_index("i")
        left = lax.rem(my_pos + N_DEV - 1, N_DEV)
        right = lax.rem(my_pos + 1, N_DEV)

        barrier_sem = pltpu.get_barrier_semaphore()
        for nbr in (left, right):
            pl.semaphore_signal(barrier_sem, inc=1, device_id=(nbr,),
                                device_id_type=pl.DeviceIdType.MESH)
        pl.semaphore_wait(barrier_sem, 2)

        xv = x_ref[:, :]

        scores = jnp.dot(xv, router_ref[:, :],
                         preferred_element_type=jnp.float32,
                         precision=lax.Precision.HIGHEST)
        scores = scores - jnp.max(scores, axis=-1, keepdims=True)
        ex = jnp.exp(scores)
        probs = ex / jnp.sum(ex, axis=-1, keepdims=True)

        lanes = lax.broadcasted_iota(jnp.int32, (n, N_EXPERTS), 1)
        gate = probs * (lanes == idx_ref[:, :]).astype(jnp.float32)
        r16 = lax.broadcasted_iota(jnp.int32, (N_EXPERTS, N_LOCAL_E), 0)
        c4 = lax.broadcasted_iota(jnp.int32, (N_EXPERTS, N_LOCAL_E), 1)
        sel = (r16 == N_LOCAL_E * my_pos + c4).astype(jnp.float32)
        coeff = jnp.dot(gate, sel, preferred_element_type=jnp.float32)

        acc = jnp.zeros((n, h), jnp.float32)
        for el in range(N_LOCAL_E):
            y = jnp.dot(xv, expW_ref[el], preferred_element_type=jnp.float32)
            acc = acc + coeff[:, el:el + 1] * y
        partial_ref[:, :] = acc

        start_chunk = lax.rem(my_pos + N_DEV - 1, N_DEV)
        comm_ref[0, :, :] = partial_ref[pl.ds(start_chunk * chunk, chunk), :]

        for hop in range(N_DEV - 1):
            send_slot = hop % 2
            recv_slot = (hop + 1) % 2
            rdma = pltpu.make_async_remote_copy(
                src_ref=comm_ref.at[send_slot],
                dst_ref=comm_ref.at[recv_slot],
                send_sem=send_sems.at[send_slot],
                recv_sem=recv_sems.at[recv_slot],
                device_id=(right,),
                device_id_type=pl.DeviceIdType.MESH,
            )
            rdma.start()
            rdma.wait()

            rc = lax.rem(my_pos + 2 * N_DEV - 2 - hop, N_DEV)
            if hop < N_DEV - 2:
                comm_ref[recv_slot, :, :] = (
                    comm_ref[recv_slot, :, :]
                    + partial_ref[pl.ds(rc * chunk, chunk), :]
                )
            else:
                shared_chunk = jnp.dot(
                    x_ref[pl.ds(my_pos * chunk, chunk), :],
                    sharedW_ref[:, :],
                    preferred_element_type=jnp.float32,
                )
                out_ref[:, :] = (
                    comm_ref[recv_slot, :, :]
                    + partial_ref[pl.ds(my_pos * chunk, chunk), :]
                    + shared_chunk
                )

    return pl.pallas_call(
        body,
        out_shape=jax.ShapeDtypeStruct((chunk, h), jnp.float32),
        in_specs=[pl.BlockSpec(memory_space=pltpu.VMEM)] * 5,
        out_specs=pl.BlockSpec(memory_space=pltpu.VMEM),
        scratch_shapes=[
            pltpu.VMEM((n, h), jnp.float32),
            pltpu.VMEM((2, chunk, h), jnp.float32),
            pltpu.SemaphoreType.DMA((2,)),
            pltpu.SemaphoreType.DMA((2,)),
        ],
        compiler_params=pltpu.CompilerParams(collective_id=0),
    )(x, router_W, route_idx, expert_W, shared_W)


# device time: 43009 ns/iter; 1.4526x vs baseline; 1.4526x over previous
import jax
import jax.numpy as jnp
from jax import lax
from jax.experimental import pallas as pl
from jax.experimental.pallas import tpu as pltpu

N_DEV = 4
N_EXPERTS = 16
N_LOCAL_E = 4


def kernel(x, router_W, route_idx, expert_W, shared_W):
    n, d = x.shape
    _, _, h = expert_W.shape
    chunk = n // N_DEV
    half = h // 2

    def body(x_ref, router_ref, idx_ref, expW_ref, sharedW_ref,
             out_ref, partial_ref, coeff_ref,
             comm_r, comm_l, send_r, recv_r, send_l, recv_l):
        my_pos = lax.axis_index("i")
        left = lax.rem(my_pos + N_DEV - 1, N_DEV)
        right = lax.rem(my_pos + 1, N_DEV)

        barrier_sem = pltpu.get_barrier_semaphore()
        for nbr in (left, right):
            pl.semaphore_signal(barrier_sem, inc=1, device_id=(nbr,),
                                device_id_type=pl.DeviceIdType.MESH)
        pl.semaphore_wait(barrier_sem, 2)

        xv = x_ref[:, :]
        scores = jnp.dot(xv, router_ref[:, :],
                         preferred_element_type=jnp.float32,
                         precision=lax.Precision.HIGHEST)
        scores = scores - jnp.max(scores, axis=-1, keepdims=True)
        ex = jnp.exp(scores)
        probs = ex / jnp.sum(ex, axis=-1, keepdims=True)
        lanes = lax.broadcasted_iota(jnp.int32, (n, N_EXPERTS), 1)
        gate = probs * (lanes == idx_ref[:, :]).astype(jnp.float32)
        r16 = lax.broadcasted_iota(jnp.int32, (N_EXPERTS, N_LOCAL_E), 0)
        c4 = lax.broadcasted_iota(jnp.int32, (N_EXPERTS, N_LOCAL_E), 1)
        sel = (r16 == N_LOCAL_E * my_pos + c4).astype(jnp.float32)
        coeff_ref[:, :] = jnp.dot(gate, sel, preferred_element_type=jnp.float32)

        def compute_chunk(c):
            xc = x_ref[pl.ds(c * chunk, chunk), :]
            cf = coeff_ref[pl.ds(c * chunk, chunk), :]
            acc = jnp.zeros((chunk, h), jnp.float32)
            for el in range(N_LOCAL_E):
                y = jnp.dot(xc, expW_ref[el], preferred_element_type=jnp.float32)
                acc = acc + cf[:, el:el + 1] * y
            partial_ref[pl.ds(c * chunk, chunk), :] = acc

        def hop(direction_right, hop_idx):
            send_slot = hop_idx % 2
            recv_slot = (hop_idx + 1) % 2
            cm = comm_r if direction_right else comm_l
            ss = send_r if direction_right else send_l
            rs = recv_r if direction_right else recv_l
            tgt = right if direction_right else left
            return pltpu.make_async_remote_copy(
                src_ref=cm.at[send_slot],
                dst_ref=cm.at[recv_slot],
                send_sem=ss.at[send_slot],
                recv_sem=rs.at[recv_slot],
                device_id=(tgt,),
                device_id_type=pl.DeviceIdType.MESH,
            )

        cA = lax.rem(my_pos + N_DEV - 1, N_DEV)
        cB = lax.rem(my_pos + 1, N_DEV)
        cC = lax.rem(my_pos + 2, N_DEV)
        cD = my_pos

        compute_chunk(cA)
        compute_chunk(cB)
        comm_r[0, :, :] = partial_ref[pl.ds(cA * chunk, chunk), pl.ds(0, half)]
        comm_l[0, :, :] = partial_ref[pl.ds(cB * chunk, chunk), pl.ds(half, half)]
        r0 = hop(True, 0)
        l0 = hop(False, 0)
        r0.start()
        l0.start()

        compute_chunk(cC)
        r0.wait()
        l0.wait()
        comm_r[1, :, :] = comm_r[1, :, :] + partial_ref[
            pl.ds(cC * chunk, chunk), pl.ds(0, half)]
        comm_l[1, :, :] = comm_l[1, :, :] + partial_ref[
            pl.ds(cC * chunk, chunk), pl.ds(half, half)]
        r1 = hop(True, 1)
        l1 = hop(False, 1)
        r1.start()
        l1.start()

        compute_chunk(cD)
        shared_chunk = jnp.dot(
            x_ref[pl.ds(my_pos * chunk, chunk), :], sharedW_ref[:, :],
            preferred_element_type=jnp.float32,
        )
        r1.wait()
        l1.wait()
        comm_r[0, :, :] = comm_r[0, :, :] + partial_ref[
            pl.ds(cB * chunk, chunk), pl.ds(0, half)]
        comm_l[0, :, :] = comm_l[0, :, :] + partial_ref[
            pl.ds(cA * chunk, chunk), pl.ds(half, half)]
        r2 = hop(True, 2)
        l2 = hop(False, 2)
        r2.start()
        l2.start()

        own = partial_ref[pl.ds(cD * chunk, chunk), :] + shared_chunk

        r2.wait()
        l2.wait()
        out_ref[:, pl.ds(0, half)] = comm_r[1, :, :] + own[:, :half]
        out_ref[:, pl.ds(half, half)] = comm_l[1, :, :] + own[:, half:]

    return pl.pallas_call(
        body,
        out_shape=jax.ShapeDtypeStruct((chunk, h), jnp.float32),
        in_specs=[pl.BlockSpec(memory_space=pltpu.VMEM)] * 5,
        out_specs=pl.BlockSpec(memory_space=pltpu.VMEM),
        scratch_shapes=[
            pltpu.VMEM((n, h), jnp.float32),
            pltpu.VMEM((n, N_LOCAL_E), jnp.float32),
            pltpu.VMEM((2, chunk, half), jnp.float32),
            pltpu.VMEM((2, chunk, half), jnp.float32),
            pltpu.SemaphoreType.DMA((2,)),
            pltpu.SemaphoreType.DMA((2,)),
            pltpu.SemaphoreType.DMA((2,)),
            pltpu.SemaphoreType.DMA((2,)),
        ],
        compiler_params=pltpu.CompilerParams(collective_id=0),
    )(x, router_W, route_idx, expert_W, shared_W)


# device time: 34794 ns/iter; 1.7956x vs baseline; 1.2361x over previous
import jax
import jax.numpy as jnp
from jax import lax
from jax.experimental import pallas as pl
from jax.experimental.pallas import tpu as pltpu

N_DEV = 4
N_EXPERTS = 16
N_LOCAL_E = 4


def kernel(x, router_W, route_idx, expert_W, shared_W):
    n, d = x.shape
    _, _, h = expert_W.shape
    chunk = n // N_DEV
    half = h // 2

    def body(x_ref, router_ref, idx_ref, expW_ref, sharedW_ref,
             out_ref, partial_ref, coeff_ref,
             comm_r, comm_l, send_r, recv_r, send_l, recv_l):
        my_pos = lax.axis_index("i")
        left = lax.rem(my_pos + N_DEV - 1, N_DEV)
        right = lax.rem(my_pos + 1, N_DEV)

        barrier_sem = pltpu.get_barrier_semaphore()
        for nbr in (left, right):
            pl.semaphore_signal(barrier_sem, inc=1, device_id=(nbr,),
                                device_id_type=pl.DeviceIdType.MESH)

        xv = x_ref[:, :]
        scores = jnp.dot(xv, router_ref[:, :],
                         preferred_element_type=jnp.float32,
                         precision=lax.Precision.HIGHEST)
        scores = scores - jnp.max(scores, axis=-1, keepdims=True)
        ex = jnp.exp(scores)
        probs = ex / jnp.sum(ex, axis=-1, keepdims=True)
        lanes = lax.broadcasted_iota(jnp.int32, (n, N_EXPERTS), 1)
        gate = probs * (lanes == idx_ref[:, :]).astype(jnp.float32)
        r16 = lax.broadcasted_iota(jnp.int32, (N_EXPERTS, N_LOCAL_E), 0)
        c4 = lax.broadcasted_iota(jnp.int32, (N_EXPERTS, N_LOCAL_E), 1)
        sel = (r16 == N_LOCAL_E * my_pos + c4).astype(jnp.float32)
        coeff_ref[:, :] = jnp.dot(gate, sel, preferred_element_type=jnp.float32)

        def half_partial(c, col0):
            xc = x_ref[pl.ds(c * chunk, chunk), :]
            cf = coeff_ref[pl.ds(c * chunk, chunk), :]
            acc = jnp.zeros((chunk, half), jnp.float32)
            for el in range(N_LOCAL_E):
                w = expW_ref[el, :, pl.ds(col0, half)]
                y = jnp.dot(xc, w, preferred_element_type=jnp.float32)
                acc = acc + cf[:, el:el + 1] * y
            return acc

        def compute_half(c, col0):
            partial_ref[pl.ds(c * chunk, chunk), pl.ds(col0, half)] = (
                half_partial(c, col0))

        def hop(direction_right, hop_idx):
            send_slot = hop_idx % 2
            recv_slot = (hop_idx + 1) % 2
            cm = comm_r if direction_right else comm_l
            ss = send_r if direction_right else send_l
            rs = recv_r if direction_right else recv_l
            tgt = right if direction_right else left
            return pltpu.make_async_remote_copy(
                src_ref=cm.at[send_slot],
                dst_ref=cm.at[recv_slot],
                send_sem=ss.at[send_slot],
                recv_sem=rs.at[recv_slot],
                device_id=(tgt,),
                device_id_type=pl.DeviceIdType.MESH,
            )

        cA = lax.rem(my_pos + N_DEV - 1, N_DEV)
        cB = lax.rem(my_pos + 1, N_DEV)
        cC = lax.rem(my_pos + 2, N_DEV)
        cD = my_pos

        comm_r[0, :, :] = half_partial(cA, 0)
        comm_l[0, :, :] = half_partial(cB, half)

        pl.semaphore_wait(barrier_sem, 2)
        r0 = hop(True, 0)
        l0 = hop(False, 0)
        r0.start()
        l0.start()

        compute_half(cC, 0)
        compute_half(cC, half)
        r0.wait()
        l0.wait()
        comm_r[1, :, :] = comm_r[1, :, :] + partial_ref[
            pl.ds(cC * chunk, chunk), pl.ds(0, half)]
        comm_l[1, :, :] = comm_l[1, :, :] + partial_ref[
            pl.ds(cC * chunk, chunk), pl.ds(half, half)]
        r1 = hop(True, 1)
        l1 = hop(False, 1)
        r1.start()
        l1.start()

        compute_half(cB, 0)
        compute_half(cA, half)
        r1.wait()
        l1.wait()
        comm_r[0, :, :] = comm_r[0, :, :] + partial_ref[
            pl.ds(cB * chunk, chunk), pl.ds(0, half)]
        comm_l[0, :, :] = comm_l[0, :, :] + partial_ref[
            pl.ds(cA * chunk, chunk), pl.ds(half, half)]
        r2 = hop(True, 2)
        l2 = hop(False, 2)
        r2.start()
        l2.start()

        shared_chunk = jnp.dot(
            x_ref[pl.ds(cD * chunk, chunk), :], sharedW_ref[:, :],
            preferred_element_type=jnp.float32,
        )
        own_r = half_partial(cD, 0) + shared_chunk[:, :half]
        own_l = half_partial(cD, half) + shared_chunk[:, half:]

        r2.wait()
        l2.wait()
        out_ref[:, pl.ds(0, half)] = comm_r[1, :, :] + own_r
        out_ref[:, pl.ds(half, half)] = comm_l[1, :, :] + own_l

    return pl.pallas_call(
        body,
        out_shape=jax.ShapeDtypeStruct((chunk, h), jnp.float32),
        in_specs=[pl.BlockSpec(memory_space=pltpu.VMEM)] * 5,
        out_specs=pl.BlockSpec(memory_space=pltpu.VMEM),
        scratch_shapes=[
            pltpu.VMEM((n, h), jnp.float32),
            pltpu.VMEM((n, N_LOCAL_E), jnp.float32),
            pltpu.VMEM((2, chunk, half), jnp.float32),
            pltpu.VMEM((2, chunk, half), jnp.float32),
            pltpu.SemaphoreType.DMA((2,)),
            pltpu.SemaphoreType.DMA((2,)),
            pltpu.SemaphoreType.DMA((2,)),
            pltpu.SemaphoreType.DMA((2,)),
        ],
        compiler_params=pltpu.CompilerParams(collective_id=0),
    )(x, router_W, route_idx, expert_W, shared_W)


# device time: 20273 ns/iter; 3.0818x vs baseline; 1.7163x over previous
import jax
import jax.numpy as jnp
from jax import lax
from jax.experimental import pallas as pl
from jax.experimental.pallas import tpu as pltpu

N_DEV = 4
N_EXPERTS = 16
N_LOCAL_E = 4


def kernel(x, router_W, route_idx, expert_W, shared_W):
    n, d = x.shape
    _, _, h = expert_W.shape
    chunk = n // N_DEV
    half = h // 2

    def body(x_ref, router_ref, idx_ref, expW_ref, sharedW_ref,
             out_ref, partial_ref, coeff_ref,
             comm_r, comm_l, send_r, recv_r, send_l, recv_l):
        my_pos = lax.axis_index("i")
        left = lax.rem(my_pos + N_DEV - 1, N_DEV)
        right = lax.rem(my_pos + 1, N_DEV)

        barrier_sem = pltpu.get_barrier_semaphore()
        for nbr in (left, right):
            pl.semaphore_signal(barrier_sem, inc=1, device_id=(nbr,),
                                device_id_type=pl.DeviceIdType.MESH)

        xv = x_ref[:, :]
        scores = jnp.dot(xv, router_ref[:, :],
                         preferred_element_type=jnp.float32,
                         precision=lax.Precision.HIGHEST)
        scores = scores - jnp.max(scores, axis=-1, keepdims=True)
        ex = jnp.exp(scores)
        probs = ex / jnp.sum(ex, axis=-1, keepdims=True)
        lanes = lax.broadcasted_iota(jnp.int32, (n, N_EXPERTS), 1)
        gate = probs * (lanes == idx_ref[:, :]).astype(jnp.float32)
        r16 = lax.broadcasted_iota(jnp.int32, (N_EXPERTS, N_LOCAL_E), 0)
        c4 = lax.broadcasted_iota(jnp.int32, (N_EXPERTS, N_LOCAL_E), 1)
        sel = (r16 == N_LOCAL_E * my_pos + c4).astype(jnp.float32)
        coeff_ref[:, :] = jnp.dot(gate, sel, preferred_element_type=jnp.float32)

        def half_partial(c, col0):
            xc = x_ref[pl.ds(c * chunk, chunk), :]
            cf = coeff_ref[pl.ds(c * chunk, chunk), :]
            acc = jnp.zeros((chunk, half), jnp.float32)
            for el in range(N_LOCAL_E):
                w = expW_ref[el, :, pl.ds(col0, half)]
                y = jnp.dot(xc, w, preferred_element_type=jnp.float32)
                acc = acc + cf[:, el:el + 1] * y
            return acc

        def compute_half(c, col0):
            partial_ref[pl.ds(c * chunk, chunk), pl.ds(col0, half)] = (
                half_partial(c, col0))

        def hop(direction_right, hop_idx):
            send_slot = hop_idx % 2
            recv_slot = (hop_idx + 1) % 2
            cm = comm_r if direction_right else comm_l
            ss = send_r if direction_right else send_l
            rs = recv_r if direction_right else recv_l
            tgt = right if direction_right else left
            return pltpu.make_async_remote_copy(
                src_ref=cm.at[send_slot],
                dst_ref=cm.at[recv_slot],
                send_sem=ss.at[send_slot],
                recv_sem=rs.at[recv_slot],
                device_id=(tgt,),
                device_id_type=pl.DeviceIdType.MESH,
            )

        cA = lax.rem(my_pos + N_DEV - 1, N_DEV)
        cB = lax.rem(my_pos + 1, N_DEV)
        cC = lax.rem(my_pos + 2, N_DEV)
        cD = my_pos

        comm_r[0, :, :] = half_partial(cA, 0).astype(jnp.bfloat16)
        comm_l[0, :, :] = half_partial(cB, half).astype(jnp.bfloat16)

        pl.semaphore_wait(barrier_sem, 2)
        r0 = hop(True, 0)
        l0 = hop(False, 0)
        r0.start()
        l0.start()

        compute_half(cC, 0)
        compute_half(cC, half)
        r0.wait()
        l0.wait()
        comm_r[1, :, :] = (
            comm_r[1, :, :].astype(jnp.float32)
            + partial_ref[pl.ds(cC * chunk, chunk), pl.ds(0, half)]
        ).astype(jnp.bfloat16)
        comm_l[1, :, :] = (
            comm_l[1, :, :].astype(jnp.float32)
            + partial_ref[pl.ds(cC * chunk, chunk), pl.ds(half, half)]
        ).astype(jnp.bfloat16)
        r1 = hop(True, 1)
        l1 = hop(False, 1)
        r1.start()
        l1.start()

        compute_half(cB, 0)
        compute_half(cA, half)
        r1.wait()
        l1.wait()
        comm_r[0, :, :] = (
            comm_r[0, :, :].astype(jnp.float32)
            + partial_ref[pl.ds(cB * chunk, chunk), pl.ds(0, half)]
        ).astype(jnp.bfloat16)
        comm_l[0, :, :] = (
            comm_l[0, :, :].astype(jnp.float32)
            + partial_ref[pl.ds(cA * chunk, chunk), pl.ds(half, half)]
        ).astype(jnp.bfloat16)
        r2 = hop(True, 2)
        l2 = hop(False, 2)
        r2.start()
        l2.start()

        shared_chunk = jnp.dot(
            x_ref[pl.ds(cD * chunk, chunk), :], sharedW_ref[:, :],
            preferred_element_type=jnp.float32,
        )
        own_r = half_partial(cD, 0) + shared_chunk[:, :half]
        own_l = half_partial(cD, half) + shared_chunk[:, half:]

        r2.wait()
        l2.wait()
        out_ref[:, pl.ds(0, half)] = comm_r[1, :, :].astype(jnp.float32) + own_r
        out_ref[:, pl.ds(half, half)] = comm_l[1, :, :].astype(jnp.float32) + own_l

    return pl.pallas_call(
        body,
        out_shape=jax.ShapeDtypeStruct((chunk, h), jnp.float32),
        in_specs=[pl.BlockSpec(memory_space=pltpu.VMEM)] * 5,
        out_specs=pl.BlockSpec(memory_space=pltpu.VMEM),
        scratch_shapes=[
            pltpu.VMEM((n, h), jnp.float32),
            pltpu.VMEM((n, N_LOCAL_E), jnp.float32),
            pltpu.VMEM((2, chunk, half), jnp.bfloat16),
            pltpu.VMEM((2, chunk, half), jnp.bfloat16),
            pltpu.SemaphoreType.DMA((2,)),
            pltpu.SemaphoreType.DMA((2,)),
            pltpu.SemaphoreType.DMA((2,)),
            pltpu.SemaphoreType.DMA((2,)),
        ],
        compiler_params=pltpu.CompilerParams(collective_id=0),
    )(x, router_W, route_idx, expert_W, shared_W)
